# baseline (speedup 1.0000x reference)
import jax
import jax.numpy as jnp
from jax import lax
from jax.experimental import pallas as pl
from jax.experimental.pallas import tpu as pltpu
from jax.experimental.pallas import tpu_sc as plsc

TOTAL = 32768
D = 256
N_OUT = TOTAL - 2
RS_OUT = TOTAL - 1
BLK = 14928
NC = 2
NS = 16
NW = NC * NS
RCH = 1024


def _tc_data_body(x_ref, data_ref):
    data_ref[...] = x_ref[...]


def _sc_rs_body(rs_hbm, rs_out, rs_v):
    c = lax.axis_index("c")
    s = lax.axis_index("s")
    wid = s * NC + c
    base = wid * RCH

    @pl.when(wid < NW - 1)
    def _():
        pltpu.sync_copy(rs_hbm.at[pl.ds(base, RCH)], rs_v.at[pl.ds(0, RCH)])
        pltpu.sync_copy(rs_v.at[pl.ds(0, RCH)], rs_out.at[pl.ds(base, RCH)])

    @pl.when(wid == NW - 1)
    def _():
        tb = (NW - 1) * RCH
        pltpu.sync_copy(rs_hbm.at[pl.ds(tb, RCH - 1)], rs_v.at[pl.ds(0, RCH - 1)])
        pltpu.sync_copy(rs_v.at[pl.ds(0, RCH - 1)], rs_out.at[pl.ds(tb, RCH - 1)])


def kernel(x_data, x_row_splits):
    mesh = plsc.VectorSubcoreMesh(core_axis_name="c", subcore_axis_name="s")
    rs = pl.kernel(
        _sc_rs_body,
        mesh=mesh,
        out_type=jax.ShapeDtypeStruct((RS_OUT,), jnp.int32),
        scratch_types=[pltpu.VMEM((RCH,), jnp.int32)],
    )(x_row_splits)

    data = pl.pallas_call(
        _tc_data_body,
        grid=(pl.cdiv(N_OUT, BLK),),
        in_specs=[pl.BlockSpec((BLK, D), lambda i: (i, 0))],
        out_specs=pl.BlockSpec((BLK, D), lambda i: (i, 0)),
        out_shape=jax.ShapeDtypeStruct((N_OUT, D), jnp.float32),
    )(x_data)
    return (data, rs)
